# baseline (speedup 1.0000x reference)
import functools
import jax
import jax.numpy as jnp
from jax import lax
from jax.experimental import pallas as pl
from jax.experimental.pallas import tpu as pltpu
from jax.experimental.pallas import tpu_sc as plsc

_DELTA_VAR = 0.5
_DELTA_DIST = 1.5
_ALPHA = 1.0
_BETA = 1.0
_GAMMA = 0.1

_B, _C, _P = 4, 4, 512 * 512
_NW_PER_B = 8
_CHUNK = _P // _NW_PER_B
_SUB = 8192
_NSUB = _CHUNK // _SUB


def _sqrt16(x):
    xs = jnp.maximum(x, jnp.float32(1e-30))
    i = lax.bitcast_convert_type(xs, jnp.int32)
    i = jnp.int32(0x5F3759DF) - lax.shift_right_logical(i, 1)
    r = lax.bitcast_convert_type(i, jnp.float32)
    for _ in range(3):
        r = r * (1.5 - 0.5 * xs * r * r)
    return xs * r


def _sc_body(emb_hbm, mask_hbm, out_hbm,
             m_buf, e_buf, acc48, tacc, comb48, meanb, row64, shared, shared2):
    cc = lax.axis_index("c")
    ss = lax.axis_index("s")
    group = ss // _NW_PER_B
    batch = 2 * cc + group
    chunk = ss % _NW_PER_B
    base = chunk * _CHUNK

    iota = lax.iota(jnp.int32, 16)
    zeros16 = jnp.zeros((16,), jnp.float32)
    ones16 = jnp.ones((16,), jnp.float32)

    for j in range(3):
        acc48[pl.ds(j * 16, 16)] = zeros16
    tacc[...] = zeros16

    for sub in range(_NSUB):
        lo = base + sub * _SUB
        pltpu.sync_copy(mask_hbm.at[batch, pl.ds(lo, _SUB)], m_buf)
        for c in range(_C):
            pltpu.sync_copy(emb_hbm.at[batch, c, pl.ds(lo, _SUB)],
                            e_buf.at[c])

        def p1(i, _):
            off = pl.multiple_of(i * 16, 16)
            m16 = m_buf[pl.ds(off, 16)]
            ev = [e_buf[c, pl.ds(off, 16)] for c in range(_C)]
            nsq = ev[0] * ev[0] + ev[1] * ev[1] + ev[2] * ev[2] + ev[3] * ev[3]
            nrm = _sqrt16(nsq)
            for c in range(_C):
                plsc.addupdate_scatter(acc48, [m16 + (8 * c)], ev[c])
            plsc.addupdate_scatter(acc48, [m16 + 32], ones16)
            plsc.addupdate_scatter(acc48, [m16 + 40], nrm)
            return 0

        lax.fori_loop(0, _SUB // 16, p1, 0)

    row64[pl.ds(0, 16)] = acc48[pl.ds(0, 16)]
    row64[pl.ds(16, 16)] = acc48[pl.ds(16, 16)]
    row64[pl.ds(32, 16)] = acc48[pl.ds(32, 16)]
    row64[pl.ds(48, 16)] = zeros16
    pltpu.sync_copy(row64.at[pl.ds(0, 64)], shared.at[ss])
    plsc.subcore_barrier()

    a0 = zeros16
    a1 = zeros16
    a2 = zeros16
    for r in range(_NW_PER_B):
        pltpu.sync_copy(shared.at[group * _NW_PER_B + r], row64.at[pl.ds(0, 64)])
        a0 = a0 + row64[pl.ds(0, 16)]
        a1 = a1 + row64[pl.ds(16, 16)]
        a2 = a2 + row64[pl.ds(32, 16)]
    comb48[pl.ds(0, 16)] = a0
    comb48[pl.ds(16, 16)] = a1
    comb48[pl.ds(32, 16)] = a2

    lane8 = jnp.bitwise_and(iota, jnp.int32(7))
    cnt_rep = plsc.load_gather(comb48, [lane8 + 32])
    inv = ones16 / jnp.maximum(cnt_rep, 1.0)
    meanb[pl.ds(0, 16)] = a0 * inv
    meanb[pl.ds(16, 16)] = a1 * inv

    for sub in range(_NSUB):
        lo = base + sub * _SUB
        pltpu.sync_copy(mask_hbm.at[batch, pl.ds(lo, _SUB)], m_buf)
        for c in range(_C):
            pltpu.sync_copy(emb_hbm.at[batch, c, pl.ds(lo, _SUB)],
                            e_buf.at[c])

        def p2(i, _):
            off = pl.multiple_of(i * 16, 16)
            m16 = m_buf[pl.ds(off, 16)]
            dsq = zeros16
            for c in range(_C):
                ev = e_buf[c, pl.ds(off, 16)]
                mv = plsc.load_gather(meanb, [m16 + (8 * c)])
                d = ev - mv
                dsq = dsq + d * d
            plsc.addupdate_scatter(tacc, [m16], _sqrt16(dsq))
            return 0

        lax.fori_loop(0, _SUB // 16, p2, 0)

    pltpu.sync_copy(tacc, shared2.at[ss])
    plsc.subcore_barrier()

    @pl.when(chunk == 0)
    def _leader():
        t16 = zeros16
        for r in range(_NW_PER_B):
            pltpu.sync_copy(shared2.at[group * _NW_PER_B + r], tacc)
            t16 = t16 + tacc[...]

        def bc(x):
            return jnp.full((16,), x, jnp.float32)

        cnt = plsc.load_gather(comb48, [lane8 + 32])
        nrms = plsc.load_gather(comb48, [lane8 + 40])
        lmask = (iota >= 1) & (iota < 8)
        pres = (cnt > 0.0) & lmask
        presf = jnp.where(pres, ones16, zeros16)
        ni16 = bc(jnp.sum(presf))

        safe = jnp.maximum(cnt, 1.0)
        mean_norm = t16 / safe
        tr = jnp.maximum(mean_norm - _DELTA_VAR, 0.0)
        term = tr * tr
        var16 = (bc(jnp.sum(jnp.where(pres, term, zeros16)))
                 / jnp.maximum(ni16, 1.0))

        nv16 = bc(jnp.sum(jnp.where(lmask, cnt, zeros16)))
        rs16 = bc(jnp.sum(jnp.where(lmask, nrms, zeros16)))
        reg16 = rs16 / jnp.maximum(nv16, 1.0)

        dg = jnp.float32(_DELTA_DIST) + jnp.float32(1e6)
        diag = dg * dg
        dsum16 = ni16 * diag
        for shift in range(1, 8):
            idx2 = jnp.bitwise_and(lane8 + shift, jnp.int32(7))
            psq = zeros16
            for c in range(_C):
                av = plsc.load_gather(meanb, [lane8 + (8 * c)])
                bv = plsc.load_gather(meanb, [idx2 + (8 * c)])
                d = av - bv
                psq = psq + d * d
            pd = _sqrt16(psq)
            hg = jnp.maximum(_DELTA_DIST - pd, 0.0)
            hinge = hg * hg
            cntp = plsc.load_gather(comb48, [idx2 + 32])
            okp = pres & (cntp > 0.0) & (idx2 != 0)
            dsum16 = dsum16 + bc(jnp.sum(jnp.where(okp, hinge, zeros16)))
        denom16 = ni16 * (ni16 - 1.0)
        dist16 = jnp.where(
            ni16 > 1.0, dsum16 / jnp.maximum(denom16, 1.0), zeros16)

        loss16 = _ALPHA * var16 + _BETA * dist16 + _GAMMA * reg16
        inc16 = jnp.where(nv16 > 0.0, ones16, zeros16)
        outv = jnp.where(iota == 0, loss16,
                         jnp.where(iota == 1, inc16, zeros16))
        row64[pl.ds(0, 16)] = outv
        for j in range(1, 8):
            row64[pl.ds(j * 16, 16)] = zeros16
        pltpu.sync_copy(row64, out_hbm.at[batch])


def _make_sc_kernel():
    mesh = plsc.VectorSubcoreMesh(core_axis_name="c", subcore_axis_name="s")
    return functools.partial(
        pl.kernel,
        mesh=mesh,
        out_type=jax.ShapeDtypeStruct((_B, 128), jnp.float32),
        scratch_types=[
            pltpu.VMEM((_SUB,), jnp.int32),
            pltpu.VMEM((_C, _SUB), jnp.float32),
            pltpu.VMEM((48,), jnp.float32),
            pltpu.VMEM((16,), jnp.float32),
            pltpu.VMEM((48,), jnp.float32),
            pltpu.VMEM((32,), jnp.float32),
            pltpu.VMEM((128,), jnp.float32),
            pltpu.VMEM_SHARED((16, 64), jnp.float32),
            pltpu.VMEM_SHARED((16, 16), jnp.float32),
        ],
        compiler_params=pltpu.CompilerParams(needs_layout_passes=False),
    )(_sc_body)


def kernel(embeddings, instance_mask):
    B, C, H, W = embeddings.shape
    embf = embeddings.reshape(B, C, H * W)
    maskf = instance_mask.reshape(B, H * W)
    out = _make_sc_kernel()(embf, maskf)
    losses = out[:, 0]
    incs = out[:, 1]
    n_inc = incs.sum()
    total = (losses * incs).sum() / jnp.maximum(n_inc, 1.0)
    return jnp.where(n_inc > 0, total, jnp.array(0.0, dtype=embeddings.dtype))


# speedup vs baseline: 6.5081x; 6.5081x over previous
import jax
import jax.numpy as jnp
from jax.experimental import pallas as pl
from jax.experimental.pallas import tpu as pltpu

_DELTA_VAR = 0.5
_DELTA_DIST = 1.5
_ALPHA = 1.0
_BETA = 1.0
_GAMMA = 0.1
_MAX_ID = 8


def _body(emb_ref, mask_ref, out_ref, acc_ref):
    b = pl.program_id(0)
    nb = pl.num_programs(0)

    e0 = emb_ref[0, 0]
    e1 = emb_ref[0, 1]
    e2 = emb_ref[0, 2]
    e3 = emb_ref[0, 3]
    m = mask_ref[0]

    normsq = e0 * e0 + e1 * e1 + e2 * e2 + e3 * e3
    norm = jnp.sqrt(normsq)

    validf = (m != 0).astype(jnp.float32)
    reg_sum = jnp.sum(norm * validf)

    sels = []
    self_fs = []
    cnts = []
    means = []
    for u in range(1, _MAX_ID):
        sel = m == u
        self_f = sel.astype(jnp.float32)
        cnt = jnp.sum(self_f)
        safe = jnp.maximum(cnt, 1.0)
        mu = (
            jnp.sum(self_f * e0) / safe,
            jnp.sum(self_f * e1) / safe,
            jnp.sum(self_f * e2) / safe,
            jnp.sum(self_f * e3) / safe,
        )
        sels.append(sel)
        self_fs.append(self_f)
        cnts.append(cnt)
        means.append(mu)
    n_valid = cnts[0] + cnts[1] + cnts[2] + cnts[3] + cnts[4] + cnts[5] + cnts[6]

    mc = [jnp.full_like(e0, means[6][c]) for c in range(4)]
    for u in range(1, _MAX_ID - 1):
        sel = sels[u - 1]
        mu = means[u - 1]
        for c in range(4):
            mc[c] = jnp.where(sel, mu[c], mc[c])
    d0 = e0 - mc[0]
    d1 = e1 - mc[1]
    d2 = e2 - mc[2]
    d3 = e3 - mc[3]
    dist = jnp.sqrt(d0 * d0 + d1 * d1 + d2 * d2 + d3 * d3)

    num_instances = jnp.float32(0.0)
    var_sum = jnp.float32(0.0)
    for u in range(1, _MAX_ID):
        cnt = cnts[u - 1]
        present = cnt > 0.0
        t = jnp.sum(self_fs[u - 1] * dist)
        mean_norm = t / jnp.maximum(cnt, 1.0)
        term = jnp.maximum(mean_norm - _DELTA_VAR, 0.0) ** 2
        var_sum = var_sum + jnp.where(present, term, 0.0)
        num_instances = num_instances + present.astype(jnp.float32)
    var_loss = var_sum / jnp.maximum(num_instances, 1.0)

    diag_term = jnp.maximum(jnp.float32(_DELTA_DIST) + jnp.float32(1e6), 0.0) ** 2
    dist_sum = jnp.float32(0.0)
    for u in range(_MAX_ID - 1):
        dist_sum = dist_sum + jnp.where(cnts[u] > 0.0, diag_term, 0.0)
    for u in range(_MAX_ID - 1):
        for v in range(u + 1, _MAX_ID - 1):
            mu = means[u]
            mv = means[v]
            pairsq = (
                (mu[0] - mv[0]) ** 2
                + (mu[1] - mv[1]) ** 2
                + (mu[2] - mv[2]) ** 2
                + (mu[3] - mv[3]) ** 2
            )
            pd = jnp.sqrt(pairsq)
            hinge = jnp.maximum(_DELTA_DIST - pd, 0.0) ** 2
            both = jnp.logical_and(cnts[u] > 0.0, cnts[v] > 0.0)
            dist_sum = dist_sum + 2.0 * jnp.where(both, hinge, 0.0)
    denom = num_instances * (num_instances - 1.0)
    dist_loss = jnp.where(
        num_instances > 1.0, dist_sum / jnp.maximum(denom, 1.0), 0.0
    )

    reg_loss = reg_sum / jnp.maximum(n_valid, 1.0)
    loss_b = _ALPHA * var_loss + _BETA * dist_loss + _GAMMA * reg_loss
    inc = (n_valid > 0.0).astype(jnp.float32)

    @pl.when(b == 0)
    def _init():
        acc_ref[0] = 0.0
        acc_ref[1] = 0.0

    acc_ref[0] += loss_b * inc
    acc_ref[1] += inc

    @pl.when(b == nb - 1)
    def _fin():
        s = acc_ref[0]
        n = acc_ref[1]
        total = jnp.where(n > 0.0, s / jnp.maximum(n, 1.0), 0.0)
        out_ref[:, :] = jnp.broadcast_to(total, (1, 1))


def kernel(embeddings, instance_mask):
    B, C, H, W = embeddings.shape
    out = pl.pallas_call(
        _body,
        grid=(B,),
        in_specs=[
            pl.BlockSpec((1, C, H, W), lambda b: (b, 0, 0, 0)),
            pl.BlockSpec((1, H, W), lambda b: (b, 0, 0)),
        ],
        out_specs=pl.BlockSpec((1, 1), lambda b: (0, 0)),
        out_shape=jax.ShapeDtypeStruct((1, 1), jnp.float32),
        scratch_shapes=[pltpu.SMEM((2,), jnp.float32)],
    )(embeddings, instance_mask)
    return out[0, 0]
